# baseline (speedup 1.0000x reference)
import functools

import jax
import jax.numpy as jnp
from jax import lax
from jax.experimental import pallas as pl
from jax.experimental.pallas import tpu as pltpu
from jax.experimental.pallas import tpu_sc as plsc

BATCH = 4096
SEQ = 200
NNUC = 4
DIM = 128

NW = 32
ROWS = BATCH * SEQ
RPW = ROWS // NW
UNIT = 128
NUNITS = RPW // UNIT
NRING = 4
LANES = 16


def _comb_body(nuc_ref, pos_ref, out_ref):
    pos = pos_ref[...]
    for n in range(NNUC):
        out_ref[:, n, :] = pos + nuc_ref[n, :]


def _build_comb(nucleo_emb, pos_emb):
    comb = pl.pallas_call(
        _comb_body,
        out_shape=jax.ShapeDtypeStruct((SEQ, NNUC, DIM), jnp.float32),
    )(nucleo_emb, pos_emb)
    return comb.reshape(SEQ * NNUC, DIM)


def _sc_body(x_hbm, comb_hbm, out_hbm, comb_sh, x_v,
             r0, r1, r2, r3, sg0, sg1, sg2, sg3, ss0, ss1, ss2, ss3):
    rings = (r0, r1, r2, r3)
    gsems = (sg0, sg1, sg2, sg3)
    ssems = (ss0, ss1, ss2, ss3)

    wid = lax.axis_index("s") * 2 + lax.axis_index("c")
    base0 = wid * RPW
    lane = lax.iota(jnp.int32, LANES)

    pltpu.sync_copy(x_hbm.at[pl.ds(base0, RPW)], x_v)

    @pl.when(lax.axis_index("s") == 0)
    def _stage():
        pltpu.sync_copy(comb_hbm, comb_sh)

    plsc.subcore_barrier()

    def compute_idx(k):
        for g in range(UNIT // LANES):
            off = k * UNIT + g * LANES
            r = (base0 + off) + lane
            x_v[pl.ds(off, LANES)] = (r % SEQ) * 4 + x_v[pl.ds(off, LANES)]

    def start_gather(k, u):
        compute_idx(k)
        src = comb_sh if u % 2 == 0 else comb_hbm
        pltpu.async_copy(
            src.at[x_v.at[pl.ds(k * UNIT, UNIT)]], rings[u], gsems[u]
        )

    def start_scatter(k, u):
        pltpu.async_copy(
            rings[u], out_hbm.at[pl.ds(base0 + k * UNIT, UNIT)], ssems[u]
        )

    def wait_gather(u):
        pltpu.make_async_copy(comb_hbm.at[pl.ds(0, UNIT)], rings[u],
                              gsems[u]).wait()

    def wait_scatter(u):
        pltpu.make_async_copy(rings[u], out_hbm.at[pl.ds(0, UNIT)],
                              ssems[u]).wait()

    for u in range(NRING):
        start_gather(u, u)

    def body(j, carry):
        k = j * NRING
        for u in range(NRING):
            wait_gather(u)
            start_scatter(k + u, u)
        for u in range(NRING):
            wait_scatter(u)
            start_gather(k + NRING + u, u)
        return carry

    lax.fori_loop(0, NUNITS // NRING - 1, body, 0)

    for u in range(NRING):
        wait_gather(u)
        start_scatter(NUNITS - NRING + u, u)
    for u in range(NRING):
        wait_scatter(u)


def kernel(X, nucleo_emb, pos_emb):
    comb = _build_comb(nucleo_emb, pos_emb)
    x_flat = X.reshape(ROWS)

    mesh = plsc.VectorSubcoreMesh(core_axis_name="c", subcore_axis_name="s")
    sc_embed = functools.partial(
        pl.kernel,
        mesh=mesh,
        out_type=jax.ShapeDtypeStruct((ROWS, DIM), jnp.float32),
        scratch_types=[
            pltpu.VMEM_SHARED((SEQ * NNUC, DIM), jnp.float32),
            pltpu.VMEM((RPW,), jnp.int32),
            pltpu.VMEM((UNIT, DIM), jnp.float32),
            pltpu.VMEM((UNIT, DIM), jnp.float32),
            pltpu.VMEM((UNIT, DIM), jnp.float32),
            pltpu.VMEM((UNIT, DIM), jnp.float32),
            pltpu.SemaphoreType.DMA,
            pltpu.SemaphoreType.DMA,
            pltpu.SemaphoreType.DMA,
            pltpu.SemaphoreType.DMA,
            pltpu.SemaphoreType.DMA,
            pltpu.SemaphoreType.DMA,
            pltpu.SemaphoreType.DMA,
            pltpu.SemaphoreType.DMA,
        ],
    )(_sc_body)

    out = sc_embed(x_flat, comb)
    return out.reshape(BATCH, SEQ, DIM)


# speedup vs baseline: 1.7399x; 1.7399x over previous
import functools

import jax
import jax.numpy as jnp
from jax import lax
from jax.experimental import pallas as pl
from jax.experimental.pallas import tpu as pltpu
from jax.experimental.pallas import tpu_sc as plsc

BATCH = 4096
SEQ = 200
NNUC = 4
DIM = 128

NW = 32
LP = SEQ // 2
PAIRS = BATCH * LP
PPW = PAIRS // NW
UNIT = 64
NUNITS = PPW // UNIT
NRING = 4
CROWS = LP * NNUC * NNUC


def _prep_body(nuc_ref, posp_ref, x_ref, comb_ref, idx_ref):
    p0 = posp_ref[:, 0, :]
    p1 = posp_ref[:, 1, :]
    for x0 in range(NNUC):
        a = p0 + nuc_ref[x0, :]
        for x1 in range(NNUC):
            comb_ref[:, x0, x1, 0, :] = a
    for x1 in range(NNUC):
        b = p1 + nuc_ref[x1, :]
        for x0 in range(NNUC):
            comb_ref[:, x0, x1, 1, :] = b
    lseq = lax.broadcasted_iota(jnp.int32, (SEQ, LP), 0)
    lpc = lax.broadcasted_iota(jnp.int32, (SEQ, LP), 1)
    sel = jnp.where(lseq // 2 == lpc, 1.0, 0.0).astype(jnp.float32)
    wodd = lax.broadcasted_iota(jnp.int32, (BATCH, SEQ), 1) % 2
    xw = x_ref[...].astype(jnp.float32) * jnp.where(wodd == 0, 4.0, 1.0)
    pairs = jax.lax.dot_general(
        xw, sel, (((1,), (0,)), ((), ())),
        preferred_element_type=jnp.float32,
    )
    lp = lax.broadcasted_iota(jnp.int32, (BATCH, LP), 1)
    idx_ref[...] = lp * 16 + pairs.astype(jnp.int32)


def _build_prep(nucleo_emb, pos_emb, X):
    comb2, idx = pl.pallas_call(
        _prep_body,
        out_shape=[
            jax.ShapeDtypeStruct((LP, NNUC, NNUC, 2, DIM), jnp.float32),
            jax.ShapeDtypeStruct((BATCH, LP), jnp.int32),
        ],
    )(nucleo_emb, pos_emb.reshape(LP, 2, DIM), X)
    return comb2.reshape(CROWS, 2, DIM), idx.reshape(PAIRS)


def _sc_body(idx_hbm, comb_hbm, out_hbm, comb_sh, idx_v,
             r0, r1, r2, r3, sg0, sg1, sg2, sg3, ss0, ss1, ss2, ss3):
    rings = (r0, r1, r2, r3)
    gsems = (sg0, sg1, sg2, sg3)
    ssems = (ss0, ss1, ss2, ss3)

    wid = lax.axis_index("s") * 2 + lax.axis_index("c")
    base0 = wid * PPW

    pltpu.sync_copy(idx_hbm.at[pl.ds(base0, PPW)], idx_v)

    @pl.when(lax.axis_index("s") == 0)
    def _stage():
        pltpu.sync_copy(comb_hbm, comb_sh)

    plsc.subcore_barrier()

    def start_gather(k, u):
        pltpu.async_copy(
            comb_sh.at[idx_v.at[pl.ds(k * UNIT, UNIT)]], rings[u], gsems[u]
        )

    def start_scatter(k, u):
        pltpu.async_copy(
            rings[u], out_hbm.at[pl.ds(base0 + k * UNIT, UNIT)], ssems[u]
        )

    def wait_gather(u):
        pltpu.make_async_copy(comb_hbm.at[pl.ds(0, UNIT)], rings[u],
                              gsems[u]).wait()

    def wait_scatter(u):
        pltpu.make_async_copy(rings[u], out_hbm.at[pl.ds(0, UNIT)],
                              ssems[u]).wait()

    for u in range(NRING):
        start_gather(u, u)

    def body(j, carry):
        k = j * NRING
        for u in range(NRING):
            wait_gather(u)
            start_scatter(k + u, u)
        for u in range(NRING):
            wait_scatter(u)
            start_gather(k + NRING + u, u)
        return carry

    lax.fori_loop(0, NUNITS // NRING - 1, body, 0)

    for u in range(NRING):
        wait_gather(u)
        start_scatter(NUNITS - NRING + u, u)
    for u in range(NRING):
        wait_scatter(u)


def kernel(X, nucleo_emb, pos_emb):
    comb2, idx_flat = _build_prep(nucleo_emb, pos_emb, X)

    mesh = plsc.VectorSubcoreMesh(core_axis_name="c", subcore_axis_name="s")
    sc_embed = functools.partial(
        pl.kernel,
        mesh=mesh,
        out_type=jax.ShapeDtypeStruct((PAIRS, 2, DIM), jnp.float32),
        scratch_types=[
            pltpu.VMEM_SHARED((CROWS, 2, DIM), jnp.float32),
            pltpu.VMEM((PPW,), jnp.int32),
            pltpu.VMEM((UNIT, 2, DIM), jnp.float32),
            pltpu.VMEM((UNIT, 2, DIM), jnp.float32),
            pltpu.VMEM((UNIT, 2, DIM), jnp.float32),
            pltpu.VMEM((UNIT, 2, DIM), jnp.float32),
            pltpu.SemaphoreType.DMA,
            pltpu.SemaphoreType.DMA,
            pltpu.SemaphoreType.DMA,
            pltpu.SemaphoreType.DMA,
            pltpu.SemaphoreType.DMA,
            pltpu.SemaphoreType.DMA,
            pltpu.SemaphoreType.DMA,
            pltpu.SemaphoreType.DMA,
        ],
    )(_sc_body)

    out = sc_embed(idx_flat, comb2)
    return out.reshape(BATCH, SEQ, DIM)


# speedup vs baseline: 1.7545x; 1.0084x over previous
import functools

import jax
import jax.numpy as jnp
from jax import lax
from jax.experimental import pallas as pl
from jax.experimental.pallas import tpu as pltpu
from jax.experimental.pallas import tpu_sc as plsc

BATCH = 4096
SEQ = 200
NNUC = 4
DIM = 128

NW = 32
ROWS = BATCH * SEQ
RPW = ROWS // NW
UNIT = 128
NUNITS = RPW // UNIT
NRING = 4
LANES = 16


def _comb_body(nuc_ref, pos_ref, out_ref):
    pos = pos_ref[...]
    for n in range(NNUC):
        out_ref[:, n, :] = pos + nuc_ref[n, :]


def _build_comb(nucleo_emb, pos_emb):
    comb = pl.pallas_call(
        _comb_body,
        out_shape=jax.ShapeDtypeStruct((SEQ, NNUC, DIM), jnp.float32),
    )(nucleo_emb, pos_emb)
    return comb.reshape(SEQ * NNUC, DIM)


def _sc_body(x_hbm, comb_hbm, out_hbm, comb_sh, x_v,
             r0, r1, r2, r3, sg0, sg1, sg2, sg3, ss0, ss1, ss2, ss3):
    rings = (r0, r1, r2, r3)
    gsems = (sg0, sg1, sg2, sg3)
    ssems = (ss0, ss1, ss2, ss3)

    wid = lax.axis_index("s") * 2 + lax.axis_index("c")
    base0 = wid * RPW
    lane = lax.iota(jnp.int32, LANES)

    pltpu.sync_copy(x_hbm.at[pl.ds(base0, RPW)], x_v)

    @pl.when(lax.axis_index("s") == 0)
    def _stage():
        pltpu.sync_copy(comb_hbm, comb_sh)

    plsc.subcore_barrier()

    def compute_idx(k):
        for g in range(UNIT // LANES):
            off = k * UNIT + g * LANES
            r = (base0 + off) + lane
            x_v[pl.ds(off, LANES)] = (r % SEQ) * 4 + x_v[pl.ds(off, LANES)]

    def start_gather(k, u):
        compute_idx(k)
        pltpu.async_copy(
            comb_sh.at[x_v.at[pl.ds(k * UNIT, UNIT)]], rings[u], gsems[u]
        )

    def start_scatter(k, u):
        pltpu.async_copy(
            rings[u], out_hbm.at[pl.ds(base0 + k * UNIT, UNIT)], ssems[u]
        )

    def wait_gather(u):
        pltpu.make_async_copy(comb_hbm.at[pl.ds(0, UNIT)], rings[u],
                              gsems[u]).wait()

    def wait_scatter(u):
        pltpu.make_async_copy(rings[u], out_hbm.at[pl.ds(0, UNIT)],
                              ssems[u]).wait()

    for u in range(NRING):
        start_gather(u, u)

    def body(j, carry):
        k = j * NRING
        for u in range(NRING):
            wait_gather(u)
            start_scatter(k + u, u)
        for u in range(NRING):
            wait_scatter(u)
            start_gather(k + NRING + u, u)
        return carry

    lax.fori_loop(0, NUNITS // NRING - 1, body, 0)

    for u in range(NRING):
        wait_gather(u)
        start_scatter(NUNITS - NRING + u, u)
    for u in range(NRING):
        wait_scatter(u)


def kernel(X, nucleo_emb, pos_emb):
    comb = _build_comb(nucleo_emb, pos_emb)
    x_flat = X.reshape(ROWS)

    mesh = plsc.VectorSubcoreMesh(core_axis_name="c", subcore_axis_name="s")
    sc_embed = functools.partial(
        pl.kernel,
        mesh=mesh,
        out_type=jax.ShapeDtypeStruct((ROWS, DIM), jnp.float32),
        scratch_types=[
            pltpu.VMEM_SHARED((SEQ * NNUC, DIM), jnp.float32),
            pltpu.VMEM((RPW,), jnp.int32),
            pltpu.VMEM((UNIT, DIM), jnp.float32),
            pltpu.VMEM((UNIT, DIM), jnp.float32),
            pltpu.VMEM((UNIT, DIM), jnp.float32),
            pltpu.VMEM((UNIT, DIM), jnp.float32),
            pltpu.SemaphoreType.DMA,
            pltpu.SemaphoreType.DMA,
            pltpu.SemaphoreType.DMA,
            pltpu.SemaphoreType.DMA,
            pltpu.SemaphoreType.DMA,
            pltpu.SemaphoreType.DMA,
            pltpu.SemaphoreType.DMA,
            pltpu.SemaphoreType.DMA,
        ],
    )(_sc_body)

    out = sc_embed(x_flat, comb)
    return out.reshape(BATCH, SEQ, DIM)
